# baseline (speedup 1.0000x reference)
import jax
import jax.numpy as jnp
import numpy as np
from jax.experimental import pallas as pl
from jax.experimental.pallas import tpu as pltpu

H, W = 37, 75
NT = H * W
NP = 2944
PAD = 80
NB = NP + 2 * PAD
CMAX = 512


def _build_masks() -> np.ndarray:
    t = np.arange(NP)
    w = t % W
    valid = (t < NT)
    m = np.zeros((NP, 4), np.float32)
    m[:, 0] = valid
    m[:, 1] = valid & (w != W - 1)
    m[:, 2] = valid & (w != 0)
    return m


_MASKS = _build_masks()


def _store_variants(y, cin, mv, ml, mr, s_c, s_l, s_r):
    yb = (y * mv).astype(jnp.bfloat16)
    s_c[PAD:PAD + NP, :cin] = yb
    s_l[PAD:PAD + NP, :cin] = (y * ml).astype(jnp.bfloat16)
    s_r[PAD:PAD + NP, :cin] = (y * mr).astype(jnp.bfloat16)


def _conv3x3(cin, w_ref, s_c, s_l, s_r, relu):
    acc = None
    for tap in range(9):
        di, dj = tap // 3 - 1, tap % 3 - 1
        src = s_c if dj == 0 else (s_r if dj == 1 else s_l)
        xs = src[pl.ds(PAD + di * W + dj, NP), :cin]
        y = jnp.dot(xs, w_ref[0, tap], preferred_element_type=jnp.float32)
        acc = y if acc is None else acc + y
    return jnp.maximum(acc, 0.0) if relu else acc


def _memory_body(qf_ref, keys_ref, keys_t_ref, masks_ref, w1_ref, w2_ref,
                 w3_ref, w4_ref, out_ref, s_c, s_l, s_r, tq_scratch):
    b = pl.program_id(0)

    @pl.when(b == 0)
    def _zero_pads():
        s_c[0:PAD, :] = jnp.zeros((PAD, CMAX), jnp.bfloat16)
        s_c[PAD + NP:NB, :] = jnp.zeros((PAD, CMAX), jnp.bfloat16)
        s_l[0:PAD, :] = jnp.zeros((PAD, CMAX), jnp.bfloat16)
        s_l[PAD + NP:NB, :] = jnp.zeros((PAD, CMAX), jnp.bfloat16)
        s_r[0:PAD, :] = jnp.zeros((PAD, CMAX), jnp.bfloat16)
        s_r[PAD + NP:NB, :] = jnp.zeros((PAD, CMAX), jnp.bfloat16)

    qf = qf_ref[...]
    norm = jnp.sqrt(jnp.sum(qf * qf, axis=1, keepdims=True))
    qn = qf / jnp.maximum(norm, 1e-12)

    k_norm_t = jnp.sqrt(jnp.sum(keys_t_ref[...] ** 2, axis=0, keepdims=True))
    q_norm = jnp.sqrt(jnp.sum(qn * qn, axis=1, keepdims=True))
    dots = jnp.dot(qn, keys_t_ref[...], preferred_element_type=jnp.float32)
    cos = dots / jnp.maximum(k_norm_t * q_norm, 1e-6)
    e = jnp.exp(cos - jnp.max(cos, axis=1, keepdims=True))
    score = e / jnp.sum(e, axis=1, keepdims=True)
    upd = jnp.dot(score, keys_ref[...], preferred_element_type=jnp.float32)

    mv = masks_ref[:, 0:1]
    ml = masks_ref[:, 1:2]
    mr = masks_ref[:, 2:3]

    x0 = jnp.where(b == 1, upd, qn)
    _store_variants(x0, 128, mv, ml, mr, s_c, s_l, s_r)
    y = _conv3x3(128, w1_ref, s_c, s_l, s_r, relu=True)
    _store_variants(y, 512, mv, ml, mr, s_c, s_l, s_r)
    y = _conv3x3(512, w2_ref, s_c, s_l, s_r, relu=True)
    _store_variants(y, 256, mv, ml, mr, s_c, s_l, s_r)
    y = _conv3x3(256, w3_ref, s_c, s_l, s_r, relu=True)
    _store_variants(y, 128, mv, ml, mr, s_c, s_l, s_r)
    y = _conv3x3(128, w4_ref, s_c, s_l, s_r, relu=False)

    @pl.when(b == 0)
    def _store_tq():
        tq_scratch[...] = y

    @pl.when(b == 1)
    def _combine():
        tq = tq_scratch[...]
        tk = y
        num = jnp.sum(tk * tq, axis=1, keepdims=True)
        den = jnp.maximum(
            jnp.sqrt(jnp.sum(tk * tk, axis=1, keepdims=True))
            * jnp.sqrt(jnp.sum(tq * tq, axis=1, keepdims=True)), 1e-6)
        out_ref[...] = (num / den) * qn


def _tap_weights(w_theta, w_thetak):
    o, i = w_theta.shape[0], w_theta.shape[1]
    ws = jnp.stack([w_theta, w_thetak])
    ws = jnp.transpose(ws, (0, 3, 4, 2, 1))
    return ws.reshape(2, 9, i, o).astype(jnp.bfloat16)


@jax.jit
def _run(query, keys, theta_w1, theta_w2, theta_w3, theta_w4,
         thetak_w1, thetak_w2, thetak_w3, thetak_w4):
    d = query.shape[1]
    qf = query.reshape(d, NT).T
    qf = jnp.pad(qf, ((0, NP - NT), (0, 0)))
    masks = jnp.asarray(_MASKS)
    w1 = _tap_weights(theta_w1, thetak_w1)
    w2 = _tap_weights(theta_w2, thetak_w2)
    w3 = _tap_weights(theta_w3, thetak_w3)
    w4 = _tap_weights(theta_w4, thetak_w4)

    full = lambda shape: pl.BlockSpec(shape, lambda b: (0,) * len(shape))
    per_branch = lambda shape: pl.BlockSpec((1,) + shape[1:], lambda b: (b, 0, 0, 0))

    out = pl.pallas_call(
        _memory_body,
        grid=(2,),
        in_specs=[
            full((NP, d)),
            full((256, d)),
            full((d, 256)),
            full((NP, 4)),
            per_branch(w1.shape),
            per_branch(w2.shape),
            per_branch(w3.shape),
            per_branch(w4.shape),
        ],
        out_specs=full((NP, d)),
        out_shape=jax.ShapeDtypeStruct((NP, d), jnp.float32),
        scratch_shapes=[
            pltpu.VMEM((NB, CMAX), jnp.bfloat16),
            pltpu.VMEM((NB, CMAX), jnp.bfloat16),
            pltpu.VMEM((NB, CMAX), jnp.bfloat16),
            pltpu.VMEM((NP, 64), jnp.float32),
        ],
    )(qf, keys, keys.T, masks, w1, w2, w3, w4)

    cfeature = out[:NT, :].T.reshape(1, d, H, W)
    return keys, cfeature


def kernel(query, keys, theta_w1, theta_w2, theta_w3, theta_w4,
           thetak_w1, thetak_w2, thetak_w3, thetak_w4, train=False):
    return _run(query, keys, theta_w1, theta_w2, theta_w3, theta_w4,
                thetak_w1, thetak_w2, thetak_w3, thetak_w4)


# speedup vs baseline: 1.5968x; 1.5968x over previous
import jax
import jax.numpy as jnp
import numpy as np
from jax.experimental import pallas as pl
from jax.experimental.pallas import tpu as pltpu

H, W = 37, 75
NT = H * W
NP = 2944


def _build_masks() -> np.ndarray:
    t = np.arange(NP)
    w = t % W
    valid = t < NT
    rows = [
        valid.astype(np.float32),
        (valid & (w != W - 1)).astype(np.float32),
        (valid & (w != 0)).astype(np.float32),
    ]
    rows.extend(np.zeros(NP, np.float32) for _ in range(5))
    return np.stack(rows)


_MASKS = _build_masks()


def _shift(x, delta):
    if delta == 0:
        return x
    return jnp.concatenate([x[:, delta:], x[:, :delta]], axis=1)


def _conv3x3(x_bf, w_ref, masks_ref, mvalid, relu, out_bf16):
    ml = masks_ref[pl.ds(1, 1), :].astype(jnp.bfloat16)
    mr = masks_ref[pl.ds(2, 1), :].astype(jnp.bfloat16)
    xl = x_bf * ml
    xr = x_bf * mr
    acc = None
    for tap in range(9):
        di, dj = tap // 3 - 1, tap % 3 - 1
        src = x_bf if dj == 0 else (xr if dj == 1 else xl)
        xs = _shift(src, di * W + dj)
        y = jnp.dot(w_ref[0, tap], xs, preferred_element_type=jnp.float32)
        acc = y if acc is None else acc + y
    if relu:
        acc = jnp.maximum(acc, 0.0)
    acc = acc * mvalid
    return acc.astype(jnp.bfloat16) if out_bf16 else acc


def _memory_body(qf_ref, keys_ref, masks_ref, w1_ref, w2_ref,
                 w3_ref, w4_ref, out_ref, x0_scratch, tq_scratch):
    b = pl.program_id(0)
    qt = qf_ref[...]
    qt = jnp.concatenate(
        [qt, jnp.zeros((qt.shape[0], NP - NT), jnp.float32)], axis=1)
    norm = jnp.sqrt(jnp.sum(qt * qt, axis=0, keepdims=True))
    qn = qt / jnp.maximum(norm, 1e-12)

    @pl.when(b == 0)
    def _theta_input():
        x0_scratch[...] = qn

    @pl.when(b == 1)
    def _memory_read():
        keys = keys_ref[...]
        k_norm = jnp.sqrt(jnp.sum(keys * keys, axis=1, keepdims=True))
        q_norm = jnp.sqrt(jnp.sum(qn * qn, axis=0, keepdims=True))
        dots = jnp.dot(keys, qn, preferred_element_type=jnp.float32)
        cos = dots / jnp.maximum(k_norm * q_norm, 1e-6)
        e = jnp.exp(cos - jnp.max(cos, axis=0, keepdims=True))
        score = e / jnp.sum(e, axis=0, keepdims=True)
        x0_scratch[...] = jax.lax.dot_general(
            keys, score, (((0,), (0,)), ((), ())),
            preferred_element_type=jnp.float32)

    mvalid = masks_ref[pl.ds(0, 1), :]
    x = (x0_scratch[...] * mvalid).astype(jnp.bfloat16)
    x = _conv3x3(x, w1_ref, masks_ref, mvalid, relu=True, out_bf16=True)
    x = _conv3x3(x, w2_ref, masks_ref, mvalid, relu=True, out_bf16=True)
    x = _conv3x3(x, w3_ref, masks_ref, mvalid, relu=True, out_bf16=True)
    x = _conv3x3(x, w4_ref, masks_ref, mvalid, relu=False, out_bf16=False)

    @pl.when(b == 0)
    def _store_tq():
        tq_scratch[...] = x

    @pl.when(b == 1)
    def _combine():
        tq = tq_scratch[...]
        tk = x
        num = jnp.sum(tk * tq, axis=0, keepdims=True)
        den = jnp.maximum(
            jnp.sqrt(jnp.sum(tk * tk, axis=0, keepdims=True))
            * jnp.sqrt(jnp.sum(tq * tq, axis=0, keepdims=True)), 1e-6)
        out_ref[...] = ((num / den) * qn)[:, :NT]


def _tap_weights(w_theta, w_thetak):
    o, i = w_theta.shape[0], w_theta.shape[1]
    ws = jnp.stack([w_theta, w_thetak]).astype(jnp.bfloat16)
    ws = jnp.transpose(ws, (0, 3, 4, 1, 2))
    return ws.reshape(2, 9, o, i)


@jax.jit
def _run(query, keys, theta_w1, theta_w2, theta_w3, theta_w4,
         thetak_w1, thetak_w2, thetak_w3, thetak_w4):
    d = query.shape[1]
    qf = query.reshape(d, NT)
    masks = jnp.asarray(_MASKS)
    w1 = _tap_weights(theta_w1, thetak_w1)
    w2 = _tap_weights(theta_w2, thetak_w2)
    w3 = _tap_weights(theta_w3, thetak_w3)
    w4 = _tap_weights(theta_w4, thetak_w4)

    full = lambda shape: pl.BlockSpec(shape, lambda b: (0,) * len(shape))
    per_branch = lambda shape: pl.BlockSpec((1,) + shape[1:], lambda b: (b, 0, 0, 0))

    out = pl.pallas_call(
        _memory_body,
        grid=(2,),
        in_specs=[
            full((d, NT)),
            full((256, d)),
            full((8, NP)),
            per_branch(w1.shape),
            per_branch(w2.shape),
            per_branch(w3.shape),
            per_branch(w4.shape),
        ],
        out_specs=full((d, NT)),
        out_shape=jax.ShapeDtypeStruct((d, NT), jnp.float32),
        scratch_shapes=[
            pltpu.VMEM((d, NP), jnp.float32),
            pltpu.VMEM((64, NP), jnp.float32),
        ],
    )(qf, keys, masks, w1, w2, w3, w4)

    cfeature = out.reshape(1, d, H, W)
    return keys, cfeature


def kernel(query, keys, theta_w1, theta_w2, theta_w3, theta_w4,
           thetak_w1, thetak_w2, thetak_w3, thetak_w4, train=False):
    return _run(query, keys, theta_w1, theta_w2, theta_w3, theta_w4,
                thetak_w1, thetak_w2, thetak_w3, thetak_w4)
